# Initial kernel scaffold; baseline (speedup 1.0000x reference)
import jax
import jax.numpy as jnp
from jax.experimental import pallas as pl


def kernel(features):
    raise NotImplementedError("write your pallas kernel here")


# speedup vs baseline: 1.5238x; 1.5238x over previous
import functools

import jax
import jax.numpy as jnp
from jax import lax
from jax.experimental import pallas as pl
from jax.experimental.pallas import tpu as pltpu
from jax.experimental.pallas import tpu_sc as plsc

_SIGMA = 0.1
_K = 10
_HALF = 32
_CPS = 32 * 8192
_CHUNK = 32768
_LANES = 16
_UNROLL = 8
_NC = 2
_PARTIAL_W = _K * _LANES + _LANES


def _topk_row(feat_hbm, buf, base):
    init = tuple(jnp.full((_LANES,), -jnp.inf, jnp.float32) for _ in range(_K))

    def chunk_body(ci, t):
        start = pl.multiple_of(base + ci * _CHUNK, _CHUNK)
        pltpu.sync_copy(feat_hbm.at[pl.ds(start, _CHUNK)], buf)

        def vec_body(vi, t):
            t = list(t)
            for u in range(_UNROLL):
                off = pl.multiple_of((vi * _UNROLL + u) * _LANES, _LANES)
                x = buf[pl.ds(off, _LANES)]
                for j in range(_K):
                    hi = jnp.maximum(t[j], x)
                    x = jnp.minimum(t[j], x)
                    t[j] = hi
            return tuple(t)

        return lax.fori_loop(0, _CHUNK // _LANES // _UNROLL, vec_body, t)

    return lax.fori_loop(0, _CPS // _CHUNK, chunk_body, init)


def _sum_row(feat_hbm, buf, base):
    def chunk_body(ci, acc):
        start = pl.multiple_of(base + ci * _CHUNK, _CHUNK)
        pltpu.sync_copy(feat_hbm.at[pl.ds(start, _CHUNK)], buf)

        def vec_body(vi, acc):
            for u in range(_UNROLL):
                off = pl.multiple_of((vi * _UNROLL + u) * _LANES, _LANES)
                acc = acc + buf[pl.ds(off, _LANES)]
            return acc

        return lax.fori_loop(0, _CHUNK // _LANES // _UNROLL, vec_body, acc)

    return lax.fori_loop(0, _CPS // _CHUNK, chunk_body,
                         jnp.zeros((_LANES,), jnp.float32))


@functools.partial(
    pl.kernel,
    out_type=jax.ShapeDtypeStruct((_HALF, _PARTIAL_W), jnp.float32),
    mesh=plsc.VectorSubcoreMesh(core_axis_name="c", subcore_axis_name="s"),
    scratch_types=[
        pltpu.VMEM((_CHUNK,), jnp.float32),
        pltpu.VMEM((_PARTIAL_W,), jnp.float32),
    ],
)
def _sc_partials(feat_hbm, out_hbm, buf, stage):
    wid = lax.axis_index("s") * _NC + lax.axis_index("c")
    t = _topk_row(feat_hbm, buf, (_HALF + wid) * _CPS)
    acc = _sum_row(feat_hbm, buf, wid * _CPS)
    for j in range(_K):
        stage[pl.ds(j * _LANES, _LANES)] = t[j]
    stage[pl.ds(_K * _LANES, _LANES)] = acc
    pltpu.sync_copy(stage, out_hbm.at[wid])


def _finish_body(p_ref, o_ref):
    cand = p_ref[:, : _K * _LANES]
    nor = p_ref[:, _K * _LANES:]
    iota = lax.broadcasted_iota(jnp.int32, cand.shape, 1)
    s = jnp.zeros((_HALF, 1), jnp.float32)
    for _ in range(_K):
        m = jnp.max(cand, axis=1, keepdims=True)
        s = s + m
        eq = cand == m
        first = jnp.min(jnp.where(eq, iota, jnp.int32(2**30)), axis=1,
                        keepdims=True)
        cand = jnp.where(eq & (iota == first), -jnp.inf, cand)
    loss_abn = jnp.sum(s) / (_K * _HALF)
    loss_nor = jnp.sum(nor) / (_HALF * _CPS)
    o_ref[...] = jnp.zeros((1, 1), jnp.float32) + (loss_abn - (loss_nor + _SIGMA))


def kernel(features):
    feat_flat = features.reshape(-1)
    partials = _sc_partials(feat_flat)
    out = pl.pallas_call(
        _finish_body,
        out_shape=jax.ShapeDtypeStruct((1, 1), jnp.float32),
    )(partials)
    return out[0, 0]


# speedup vs baseline: 2.0605x; 1.3522x over previous
import functools

import jax
import jax.numpy as jnp
from jax import lax
from jax.experimental import pallas as pl
from jax.experimental.pallas import tpu as pltpu
from jax.experimental.pallas import tpu_sc as plsc

_SIGMA = 0.1
_K = 10
_HALF = 32
_CPS = 32 * 8192
_CHUNK = 32768
_LANES = 16
_UNROLL = 8
_NC = 2
_PARTIAL_W = _K * _LANES + _LANES


_ROWS = _CHUNK // 8192
_VPR = 8192 // _LANES


def _topk_row(feat_hbm, buf, sample):
    init = tuple(jnp.full((_LANES,), -jnp.inf, jnp.float32) for _ in range(_K))

    def chunk_body(ci, t):
        pltpu.sync_copy(feat_hbm.at[sample, pl.ds(ci * _ROWS, _ROWS), :], buf)

        def row(r, t):
            def vec_body(vi, t):
                t = list(t)
                for u in range(_UNROLL):
                    off = pl.multiple_of((vi * _UNROLL + u) * _LANES, _LANES)
                    x = buf[r, pl.ds(off, _LANES)]
                    for j in range(_K):
                        hi = jnp.maximum(t[j], x)
                        x = jnp.minimum(t[j], x)
                        t[j] = hi
                return tuple(t)

            return lax.fori_loop(0, _VPR // _UNROLL, vec_body, t)

        for r in range(_ROWS):
            t = row(r, t)
        return t

    return lax.fori_loop(0, _CPS // _CHUNK, chunk_body, init)


def _sum_row(feat_hbm, buf, sample):
    def chunk_body(ci, acc):
        pltpu.sync_copy(feat_hbm.at[sample, pl.ds(ci * _ROWS, _ROWS), :], buf)

        def row(r, acc):
            def vec_body(vi, acc):
                for u in range(_UNROLL):
                    off = pl.multiple_of((vi * _UNROLL + u) * _LANES, _LANES)
                    acc = acc + buf[r, pl.ds(off, _LANES)]
                return acc

            return lax.fori_loop(0, _VPR // _UNROLL, vec_body, acc)

        for r in range(_ROWS):
            acc = row(r, acc)
        return acc

    return lax.fori_loop(0, _CPS // _CHUNK, chunk_body,
                         jnp.zeros((_LANES,), jnp.float32))


@functools.partial(
    pl.kernel,
    out_type=jax.ShapeDtypeStruct((_HALF, _PARTIAL_W), jnp.float32),
    mesh=plsc.VectorSubcoreMesh(core_axis_name="c", subcore_axis_name="s"),
    scratch_types=[
        pltpu.VMEM((_ROWS, 8192), jnp.float32),
        pltpu.VMEM((_PARTIAL_W,), jnp.float32),
    ],
)
def _sc_partials(feat_hbm, out_hbm, buf, stage):
    wid = lax.axis_index("s") * _NC + lax.axis_index("c")
    t = _topk_row(feat_hbm, buf, _HALF + wid)
    acc = _sum_row(feat_hbm, buf, wid)
    for j in range(_K):
        stage[pl.ds(j * _LANES, _LANES)] = t[j]
    stage[pl.ds(_K * _LANES, _LANES)] = acc
    pltpu.sync_copy(stage, out_hbm.at[wid])


def _finish_body(p_ref, o_ref):
    cand = p_ref[:, : _K * _LANES]
    nor = p_ref[:, _K * _LANES:]
    iota = lax.broadcasted_iota(jnp.int32, cand.shape, 1)
    s = jnp.zeros((_HALF, 1), jnp.float32)
    for _ in range(_K):
        m = jnp.max(cand, axis=1, keepdims=True)
        s = s + m
        eq = cand == m
        first = jnp.min(jnp.where(eq, iota, jnp.int32(2**30)), axis=1,
                        keepdims=True)
        cand = jnp.where(eq & (iota == first), -jnp.inf, cand)
    loss_abn = jnp.sum(s) / (_K * _HALF)
    loss_nor = jnp.sum(nor) / (_HALF * _CPS)
    o_ref[...] = jnp.zeros((1, 1), jnp.float32) + (loss_abn - (loss_nor + _SIGMA))


def kernel(features):
    partials = _sc_partials(features)
    out = pl.pallas_call(
        _finish_body,
        out_shape=jax.ShapeDtypeStruct((1, 1), jnp.float32),
    )(partials)
    return out[0, 0]
